# baseline (speedup 1.0000x reference)
import jax
import jax.numpy as jnp
from jax.experimental import pallas as pl

_CHUNKS = 4


def _body(x_ref, adj_ref, w_ref, out_ref):
    k = pl.program_id(1)
    n = x_ref.shape[1]
    a = adj_ref[0]
    xb = x_ref[0]
    t = jax.lax.dot_general(
        a, xb, (((0,), (0,)), ((), ())), preferred_element_type=jnp.float32
    )
    h = jnp.maximum(
        jax.lax.dot_general(
            t, w_ref[...], (((1,), (0,)), ((), ())),
            preferred_element_type=jnp.float32,
        ),
        0.0,
    )
    part = jnp.sum(h, axis=0) * (1.0 / n)

    @pl.when(k == 0)
    def _():
        out_ref[0, 0, :] = part

    @pl.when(k != 0)
    def _():
        out_ref[0, 0, :] += part


def kernel(x, adj, W):
    b, n, f_in = x.shape
    f_out = W.shape[1]
    nc = n // _CHUNKS
    return pl.pallas_call(
        _body,
        grid=(b, _CHUNKS),
        in_specs=[
            pl.BlockSpec((1, n, f_in), lambda i, k: (i, 0, 0)),
            pl.BlockSpec((1, n, nc), lambda i, k: (i, 0, k)),
            pl.BlockSpec((f_in, f_out), lambda i, k: (0, 0)),
        ],
        out_specs=pl.BlockSpec((1, 1, f_out), lambda i, k: (i, 0, 0)),
        out_shape=jax.ShapeDtypeStruct((b, 1, f_out), jnp.float32),
    )(x, adj, W).reshape(b, f_out)


# speedup vs baseline: 2.1317x; 2.1317x over previous
import jax
import jax.numpy as jnp
from jax.experimental import pallas as pl


def _body(x_ref, adj_ref, w_ref, out_ref):
    a = adj_ref[0]
    xb = x_ref[0]
    tt = jax.lax.dot_general(
        xb, a, (((0,), (0,)), ((), ())), preferred_element_type=jnp.float32
    )
    ht = jnp.maximum(
        jax.lax.dot_general(
            w_ref[...], tt, (((0,), (0,)), ((), ())),
            preferred_element_type=jnp.float32,
        ),
        0.0,
    )
    n = a.shape[0]
    out_ref[0, 0, :] = jnp.sum(ht, axis=1) * (1.0 / n)


def kernel(x, adj, W):
    b, n, f_in = x.shape
    f_out = W.shape[1]
    return pl.pallas_call(
        _body,
        grid=(b,),
        in_specs=[
            pl.BlockSpec((1, n, f_in), lambda i: (i, 0, 0)),
            pl.BlockSpec((1, n, n), lambda i: (i, 0, 0)),
            pl.BlockSpec((f_in, f_out), lambda i: (0, 0)),
        ],
        out_specs=pl.BlockSpec((1, 1, f_out), lambda i: (i, 0, 0)),
        out_shape=jax.ShapeDtypeStruct((b, 1, f_out), jnp.float32),
    )(x, adj, W).reshape(b, f_out)


# speedup vs baseline: 2.2228x; 1.0428x over previous
import jax
import jax.numpy as jnp
from jax.experimental import pallas as pl


def _body(x_ref, adj_ref, w_ref, out_ref):
    a = adj_ref[0].astype(jnp.bfloat16)
    xb = x_ref[0].astype(jnp.bfloat16)
    t = jax.lax.dot_general(
        a, xb, (((0,), (0,)), ((), ())), preferred_element_type=jnp.float32
    )
    h = jnp.maximum(
        jax.lax.dot_general(
            t, w_ref[...], (((1,), (0,)), ((), ())),
            preferred_element_type=jnp.float32,
        ),
        0.0,
    )
    n = a.shape[0]
    out_ref[0, 0, :] = jnp.sum(h, axis=0) * (1.0 / n)


def kernel(x, adj, W):
    b, n, f_in = x.shape
    f_out = W.shape[1]
    return pl.pallas_call(
        _body,
        grid=(b,),
        in_specs=[
            pl.BlockSpec((1, n, f_in), lambda i: (i, 0, 0)),
            pl.BlockSpec((1, n, n), lambda i: (i, 0, 0)),
            pl.BlockSpec((f_in, f_out), lambda i: (0, 0)),
        ],
        out_specs=pl.BlockSpec((1, 1, f_out), lambda i: (i, 0, 0)),
        out_shape=jax.ShapeDtypeStruct((b, 1, f_out), jnp.float32),
    )(x, adj, W).reshape(b, f_out)
